# baseline (speedup 1.0000x reference)
import functools

import jax
import jax.numpy as jnp
from jax import lax
from jax.experimental import pallas as pl
from jax.experimental.pallas import tpu as pltpu
from jax.experimental.pallas import tpu_sc as plsc

_CH = 128
_NW = 32


def _sc_gather(table, idx, n_chunks, n_rows_pad, d):
    iters = -(-n_chunks // _NW)
    mesh = plsc.VectorSubcoreMesh(core_axis_name="c", subcore_axis_name="s")

    @functools.partial(
        pl.kernel,
        mesh=mesh,
        compiler_params=pltpu.CompilerParams(use_tc_tiling_on_sc=False),
        out_type=jax.ShapeDtypeStruct((n_rows_pad, d), table.dtype),
        scratch_types=[
            pltpu.VMEM((2, _CH), jnp.int32),
            pltpu.VMEM((2, _CH, d), table.dtype),
            pltpu.SemaphoreType.DMA((2,)),
        ],
    )
    def gather_kernel(table_hbm, idx_hbm, out_hbm, idx_v, rows_v, sem):
        wid = lax.axis_index("s") * 2 + lax.axis_index("c")

        def start(j, slot):
            c = wid + j * _NW

            @pl.when(c < n_chunks)
            def _():
                pltpu.sync_copy(idx_hbm.at[pl.ds(c * _CH, _CH)],
                                idx_v.at[slot])
                pltpu.async_copy(table_hbm.at[idx_v.at[slot]],
                                 rows_v.at[slot], sem.at[slot])

        def finish(j, slot):
            c = wid + j * _NW

            @pl.when(c < n_chunks)
            def _():
                pltpu.make_async_copy(table_hbm.at[idx_v.at[slot]],
                                      rows_v.at[slot], sem.at[slot]).wait()
                pltpu.sync_copy(rows_v.at[slot],
                                out_hbm.at[pl.ds(c * _CH, _CH)])

        start(0, 0)

        def body(j, carry):
            slot = lax.rem(j, 2)
            start(j + 1, 1 - slot)
            finish(j, slot)
            return carry

        lax.fori_loop(0, iters, body, 0)

    return gather_kernel(table, idx)


def _tc_mlp(gathered, w1a, w1b, b1, w2, b2, w3, n, k, d, bn):
    bk = bn * k
    nblocks = n // bn
    self_block0 = (n * k) // bn

    def body(g_ref, s_ref, w1a_ref, w1b_ref, b1_ref, w2_ref, b2_ref,
             w3_ref, out_ref):
        g = g_ref[...]
        s = s_ref[...]
        u1 = jnp.dot(s, w1b_ref[...], preferred_element_type=jnp.float32)
        u1 = u1 + b1_ref[...]
        u_rep = jnp.broadcast_to(u1[:, None, :], (bn, k, d)).reshape(bk, d)
        h1 = jnp.dot(g, w1a_ref[...], preferred_element_type=jnp.float32)
        h1 = jnp.maximum(h1 + u_rep, 0.0).astype(jnp.bfloat16)
        h2 = jnp.dot(h1, w2_ref[...], preferred_element_type=jnp.float32)
        h2 = jnp.maximum(h2 + b2_ref[...], 0.0)
        t = (h2 * w3_ref[...]).reshape(bn, k, d)
        sc = jnp.sum(t, axis=2, keepdims=True)
        m = jnp.max(sc, axis=1, keepdims=True)
        e = jnp.exp(sc - m)
        att = e / jnp.sum(e, axis=1, keepdims=True)
        g3 = g.reshape(bn, k, d).astype(jnp.float32)
        agg = jnp.sum(att * g3, axis=1)
        out_ref[...] = (agg + s.astype(jnp.float32)) * 0.5

    return pl.pallas_call(
        body,
        grid=(nblocks,),
        in_specs=[
            pl.BlockSpec((bk, d), lambda b: (b, 0)),
            pl.BlockSpec((bn, d), lambda b: (self_block0 + b, 0)),
            pl.BlockSpec((d, d), lambda b: (0, 0)),
            pl.BlockSpec((d, d), lambda b: (0, 0)),
            pl.BlockSpec((1, d), lambda b: (0, 0)),
            pl.BlockSpec((d, d), lambda b: (0, 0)),
            pl.BlockSpec((1, d), lambda b: (0, 0)),
            pl.BlockSpec((1, d), lambda b: (0, 0)),
        ],
        out_specs=pl.BlockSpec((bn, d), lambda b: (b, 0)),
        out_shape=jax.ShapeDtypeStruct((n, d), jnp.float32),
        compiler_params=pltpu.CompilerParams(
            dimension_semantics=("arbitrary",)),
    )(gathered, gathered, w1a, w1b, b1, w2, b2, w3)


def kernel(nodes, neighbors, u2e_weight, W1, b1, W2, b2, W3, b3):
    n, k = neighbors.shape
    v, d = u2e_weight.shape
    total = n * k + n
    n_chunks = -(-total // _CH)
    n_rows_pad = n_chunks * _CH
    idx = jnp.concatenate([
        neighbors.reshape(-1), nodes,
        jnp.zeros((n_rows_pad - total,), jnp.int32),
    ])
    table_bf = u2e_weight.astype(jnp.bfloat16)
    table_i32 = lax.bitcast_convert_type(
        table_bf.reshape(v, d // 2, 2), jnp.int32)
    gathered_i32 = _sc_gather(table_i32, idx, n_chunks, n_rows_pad, d // 2)
    gathered = lax.bitcast_convert_type(
        gathered_i32, jnp.bfloat16).reshape(n_rows_pad, d)

    w1a = W1[:, :d].T.astype(jnp.bfloat16)
    w1b = W1[:, d:].T.astype(jnp.bfloat16)
    b1r = b1.reshape(1, d)
    w2t = W2.T.astype(jnp.bfloat16)
    b2r = b2.reshape(1, d)
    w3r = W3.reshape(1, d)
    bn = 400
    return _tc_mlp(gathered, w1a, w1b, b1r, w2t, b2r, w3r, n, k, d, bn)


# speedup vs baseline: 6.2371x; 6.2371x over previous
import functools

import jax
import jax.numpy as jnp
from jax import lax
from jax.experimental import pallas as pl
from jax.experimental.pallas import tpu as pltpu
from jax.experimental.pallas import tpu_sc as plsc

_CH = 128
_NW = 32


def _sc_gather(table, idx, n_chunks, n_rows_pad, d):
    iters = -(-n_chunks // _NW)
    mesh = plsc.VectorSubcoreMesh(core_axis_name="c", subcore_axis_name="s")

    @functools.partial(
        pl.kernel,
        mesh=mesh,
        out_type=jax.ShapeDtypeStruct((n_rows_pad, d), table.dtype),
        scratch_types=[
            pltpu.VMEM((2, _CH), jnp.int32),
            pltpu.VMEM((2, _CH, d), table.dtype),
            pltpu.SemaphoreType.DMA((2,)),
        ],
    )
    def gather_kernel(table_hbm, idx_hbm, out_hbm, idx_v, rows_v, sem):
        wid = lax.axis_index("s") * 2 + lax.axis_index("c")

        def start(j, slot):
            c = wid + j * _NW

            @pl.when(c < n_chunks)
            def _():
                pltpu.sync_copy(idx_hbm.at[pl.ds(c * _CH, _CH)],
                                idx_v.at[slot])
                pltpu.async_copy(table_hbm.at[idx_v.at[slot]],
                                 rows_v.at[slot], sem.at[slot])

        def finish(j, slot):
            c = wid + j * _NW

            @pl.when(c < n_chunks)
            def _():
                pltpu.make_async_copy(table_hbm.at[idx_v.at[slot]],
                                      rows_v.at[slot], sem.at[slot]).wait()
                pltpu.sync_copy(rows_v.at[slot],
                                out_hbm.at[pl.ds(c * _CH, _CH)])

        start(0, 0)

        def body(j, carry):
            slot = lax.rem(j, 2)
            start(j + 1, 1 - slot)
            finish(j, slot)
            return carry

        lax.fori_loop(0, iters, body, 0)

    return gather_kernel(table, idx)


def _tc_mlp(gathered, w1a, w1b, b1, w2, b2, w3, n, k, d, bn):
    bk = bn * k
    nblocks = n // bn
    self_block0 = (n * k) // bn

    def body(g_ref, s_ref, w1a_ref, w1b_ref, b1_ref, w2_ref, b2_ref,
             w3_ref, out_ref):
        g = g_ref[...]
        s = s_ref[...]
        u1 = jnp.dot(s, w1b_ref[...], preferred_element_type=jnp.float32)
        u1 = u1 + b1_ref[...]
        u_rep = jnp.broadcast_to(u1[:, None, :], (bn, k, d)).reshape(bk, d)
        h1 = jnp.dot(g, w1a_ref[...], preferred_element_type=jnp.float32)
        h1 = jnp.maximum(h1 + u_rep, 0.0)
        h2 = jnp.dot(h1, w2_ref[...], preferred_element_type=jnp.float32)
        h2 = jnp.maximum(h2 + b2_ref[...], 0.0)
        t = (h2 * w3_ref[...]).reshape(bn, k, d)
        sc = jnp.sum(t, axis=2, keepdims=True)
        m = jnp.max(sc, axis=1, keepdims=True)
        e = jnp.exp(sc - m)
        att = e / jnp.sum(e, axis=1, keepdims=True)
        g3 = g.reshape(bn, k, d)
        agg = jnp.sum(att * g3, axis=1)
        out_ref[...] = (agg + s) * 0.5

    return pl.pallas_call(
        body,
        grid=(nblocks,),
        in_specs=[
            pl.BlockSpec((bk, d), lambda b: (b, 0)),
            pl.BlockSpec((bn, d), lambda b: (self_block0 + b, 0)),
            pl.BlockSpec((d, d), lambda b: (0, 0)),
            pl.BlockSpec((d, d), lambda b: (0, 0)),
            pl.BlockSpec((1, d), lambda b: (0, 0)),
            pl.BlockSpec((d, d), lambda b: (0, 0)),
            pl.BlockSpec((1, d), lambda b: (0, 0)),
            pl.BlockSpec((1, d), lambda b: (0, 0)),
        ],
        out_specs=pl.BlockSpec((bn, d), lambda b: (b, 0)),
        out_shape=jax.ShapeDtypeStruct((n, d), jnp.float32),
        compiler_params=pltpu.CompilerParams(
            dimension_semantics=("arbitrary",)),
    )(gathered, gathered, w1a, w1b, b1, w2, b2, w3)


def kernel(nodes, neighbors, u2e_weight, W1, b1, W2, b2, W3, b3):
    n, k = neighbors.shape
    v, d = u2e_weight.shape

    w1a = W1[:, :d].T
    w1b = W1[:, d:].T
    b1r = b1.reshape(1, d)
    w2t = W2.T
    b2r = b2.reshape(1, d)
    w3r = W3.reshape(1, d)

    parts = 5
    pn = n // parts
    bn = 200
    outs = []
    for p in range(parts):
        nb = neighbors[p * pn:(p + 1) * pn].reshape(-1)
        nd = nodes[p * pn:(p + 1) * pn]
        total = pn * k + pn
        n_chunks = -(-total // _CH)
        n_rows_pad = n_chunks * _CH
        idx = jnp.concatenate([
            nb, nd, jnp.zeros((n_rows_pad - total,), jnp.int32)])
        gathered = _sc_gather(u2e_weight, idx, n_chunks, n_rows_pad, d)
        outs.append(_tc_mlp(gathered, w1a, w1b, b1r, w2t, b2r, w3r,
                            pn, k, d, bn))
    return jnp.concatenate(outs, axis=0)
